# baseline (speedup 1.0000x reference)
import functools

import jax
import jax.numpy as jnp
from jax import lax
from jax.experimental import pallas as pl
from jax.experimental.pallas import tpu as pltpu
from jax.experimental.pallas import tpu_sc as plsc

_N = 10000
_E = 320000
_D = 128
_H = 16
_C = 8

_NC = 2
_NS = 16
_NW = _NC * _NS
_CHUNK = 128
_TCHUNK = _E // _CHUNK
_MAXCW = (_TCHUNK + _NW - 1) // _NW + 1
_NBUF = 8
_OUTER = (_MAXCW + _NBUF - 1) // _NBUF
_NP = 10240
_RPS1 = _NP // _NS
_PK = _NP // 8


def _sc_mesh():
    return plsc.VectorSubcoreMesh(core_axis_name="c", subcore_axis_name="s")


def _deg_partials(ei3, ew2, zeros1):
    @functools.partial(
        pl.kernel,
        out_type=jax.ShapeDtypeStruct((_NC, _NP, _H), jnp.float32),
        mesh=_sc_mesh(),
        scratch_types=[
            pltpu.VMEM((_MAXCW, _CHUNK), jnp.int32),
            pltpu.VMEM((_MAXCW, _CHUNK), jnp.float32),
            pltpu.VMEM((_RPS1,), jnp.float32),
            pltpu.VMEM((_RPS1, _H), jnp.float32),
            pltpu.VMEM_SHARED((_NP,), jnp.float32),
            pltpu.SemaphoreType.DMA,
        ],
        compiler_params=pltpu.CompilerParams(use_tc_tiling_on_sc=False),
    )
    def k(ei_hbm, ew_hbm, z_hbm, out_hbm, col_v, ew_v, degv, expb, acc, ssem):
        cid = lax.axis_index("c")
        sid = lax.axis_index("s")
        wid = cid * _NS + sid
        lo = wid * _TCHUNK // _NW
        ncw = (wid + 1) * _TCHUNK // _NW - lo
        r0 = sid * _RPS1
        pltpu.sync_copy(z_hbm.at[pl.ds(r0, _RPS1)], acc.at[pl.ds(r0, _RPS1)])
        pltpu.sync_copy(ei_hbm.at[pl.ds(lo, _MAXCW), 1], col_v)
        pltpu.sync_copy(ew_hbm.at[pl.ds(lo, _MAXCW)], ew_v)
        plsc.subcore_barrier()

        def body(c, carry):
            pltpu.async_copy(ew_v.at[c], acc.at[col_v.at[c]], ssem, add=True)
            return carry

        lax.fori_loop(0, ncw, body, 0)

        def drain(c, carry):
            pltpu.make_async_copy(ew_v.at[0], acc.at[col_v.at[0]], ssem).wait()
            return carry

        lax.fori_loop(0, ncw, drain, 0)
        plsc.subcore_barrier()
        pltpu.sync_copy(acc.at[pl.ds(r0, _RPS1)], degv)
        for g in range(_RPS1 // 16):
            dv = degv[pl.ds(g * 16, 16)]
            for j in range(16):
                expb[g * 16 + j] = jnp.broadcast_to(dv[j], (16,))
        pltpu.sync_copy(expb, out_hbm.at[cid, pl.ds(r0, _RPS1)])

    return k(ei3, ew2, zeros1)


def _edge_aggregate(table, ei3, ew2, zeros16):
    @functools.partial(
        pl.kernel,
        out_type=jax.ShapeDtypeStruct((_NC, _NP, _H), jnp.float32),
        mesh=_sc_mesh(),
        scratch_types=[
            pltpu.VMEM((_MAXCW, _CHUNK), jnp.int32),
            pltpu.VMEM((_MAXCW, _CHUNK), jnp.int32),
            pltpu.VMEM((_MAXCW, _CHUNK), jnp.float32),
            pltpu.VMEM((_NBUF, _CHUNK, _H), jnp.float32),
            pltpu.VMEM((_NBUF, _CHUNK, _H), jnp.float32),
            pltpu.VMEM_SHARED((_NP, _H), jnp.float32),
            pltpu.SemaphoreType.DMA((_NBUF,)),
            pltpu.SemaphoreType.DMA((_NBUF,)),
        ],
        compiler_params=pltpu.CompilerParams(use_tc_tiling_on_sc=False),
    )
    def k(h_hbm, ei_hbm, ew_hbm, z_hbm, out_hbm,
          row_v, col_v, ew_v, gbuf, sbuf, acc, gsem, ssem):
        cid = lax.axis_index("c")
        sid = lax.axis_index("s")
        wid = cid * _NS + sid
        lo = wid * _TCHUNK // _NW
        ncw = (wid + 1) * _TCHUNK // _NW - lo
        r0 = sid * _RPS1
        pltpu.sync_copy(z_hbm.at[pl.ds(r0, _RPS1)], acc.at[pl.ds(r0, _RPS1)])
        pltpu.sync_copy(ei_hbm.at[pl.ds(lo, _MAXCW), 0], row_v)
        pltpu.sync_copy(ei_hbm.at[pl.ds(lo, _MAXCW), 1], col_v)
        pltpu.sync_copy(ew_hbm.at[pl.ds(lo, _MAXCW)], ew_v)
        plsc.subcore_barrier()

        def gather(c, b):
            return pltpu.make_async_copy(
                h_hbm.at[row_v.at[c]], gbuf.at[b], gsem.at[b])

        def scatter(c, b):
            return pltpu.make_async_copy(
                sbuf.at[b], acc.at[col_v.at[c]], ssem.at[b])

        for b in range(_NBUF):
            gather(b, b).start()

        def outer(o, carry):
            for b in range(_NBUF):
                c = o * _NBUF + b

                @pl.when(c < ncw)
                def _():
                    gather(c, b).wait()

                @pl.when(c + _NBUF < ncw)
                def _():
                    gather(c + _NBUF, b).start()

                @pl.when(jnp.logical_and(o > 0, c < ncw))
                def _():
                    scatter(c, b).wait()

                @pl.when(c < ncw)
                def _():
                    for g in range(_CHUNK // 16):
                        ewg = ew_v[c, pl.ds(g * 16, 16)]
                        for j in range(16):
                            e = g * 16 + j
                            sbuf[b, e] = gbuf[b, e] * ewg[j]
                    scatter(c, b).start(add=True)
            return carry

        lax.fori_loop(0, _OUTER, outer, 0)
        for b in range(_NBUF):
            scatter(b, b).wait()
        plsc.subcore_barrier()
        pltpu.sync_copy(acc.at[pl.ds(r0, _RPS1)],
                        out_hbm.at[cid, pl.ds(r0, _RPS1)])

    return k(table, ei3, ew2, zeros16)


def _tc_matmul(xp, W1bd):
    def body(x_ref, w_ref, h_ref):
        hp = jnp.dot(x_ref[...], w_ref[...], preferred_element_type=jnp.float32)
        h_ref[...] = jnp.concatenate(
            [hp, jnp.zeros((_PK - _N // 8, 128), jnp.float32)], axis=0)

    return pl.pallas_call(
        body,
        out_shape=jax.ShapeDtypeStruct((_PK, 128), jnp.float32),
    )(xp, W1bd)


def _tc_scale(dp16, h1p):
    def body(dp_ref, h_ref, dinv_ref, t1_ref, s1_ref):
        deg = dp_ref[0] + dp_ref[1] + 1.0
        dinv = jnp.where(deg > 0, lax.rsqrt(jnp.where(deg > 0, deg, 1.0)), 0.0)
        h = h_ref[...]
        dinv_ref[...] = dinv
        t1_ref[...] = h * dinv
        s1_ref[...] = h * dinv * dinv

    return pl.pallas_call(
        body,
        out_shape=[
            jax.ShapeDtypeStruct((_PK, 128), jnp.float32),
            jax.ShapeDtypeStruct((_PK, 128), jnp.float32),
            jax.ShapeDtypeStruct((_PK, 128), jnp.float32),
        ],
    )(dp16, h1p)


def _tc_mid(p1p, dinvp, s1p, b1t, Wmid):
    def body(p_ref, d_ref, s_ref, b_ref, w_ref, out_ref):
        d = d_ref[...]
        v = jnp.maximum(d * (p_ref[0] + p_ref[1]) + s_ref[...] + b_ref[...],
                        0.0)
        out_ref[...] = d * jnp.dot(v, w_ref[...],
                                   preferred_element_type=jnp.float32)

    return pl.pallas_call(
        body,
        out_shape=jax.ShapeDtypeStruct((_PK, 128), jnp.float32),
    )(p1p, dinvp, s1p, b1t, Wmid)


def _tc_combine(p2p, dinvp, t2p, b2t):
    def body(p_ref, d_ref, t_ref, b_ref, out_ref):
        out_ref[...] = (d_ref[...] * (p_ref[0] + p_ref[1] + t_ref[...])
                        + b_ref[...])

    return pl.pallas_call(
        body,
        out_shape=jax.ShapeDtypeStruct((_PK, 128), jnp.float32),
    )(p2p, dinvp, t2p, b2t)


def _tc_softmax(o):
    def body(o_ref, out_ref):
        ot = o_ref[:_N, :_C].T
        m = jnp.max(ot, axis=0, keepdims=True)
        e = jnp.exp(ot - m)
        s = jnp.sum(e, axis=0, keepdims=True)
        out_ref[...] = ot - m - jnp.log(s)

    return pl.pallas_call(
        body,
        out_shape=jax.ShapeDtypeStruct((_C, _N), jnp.float32),
    )(o).T


def kernel(x, edge_index, edge_weight, W1, b1, W2, b2):
    f32 = jnp.float32
    ei3 = edge_index.reshape(2, _TCHUNK, _CHUNK).transpose(1, 0, 2)
    ew2 = edge_weight.astype(f32).reshape(_TCHUNK, _CHUNK)
    zeros1 = jnp.zeros((_NP,), f32)
    zeros16 = jnp.zeros((_NP, _H), f32)
    eye8 = jnp.eye(8, dtype=f32)
    W1bd = jnp.kron(eye8, W1)
    Wmid = jnp.kron(eye8, jnp.pad(W2, ((0, 0), (0, _H - _C))))
    b1t = jnp.tile(b1, 8).reshape(1, 128)
    b2t = jnp.tile(jnp.pad(b2, (0, _H - _C)), 8).reshape(1, 128)

    h1p = _tc_matmul(x.reshape(_N // 8, 8 * _D), W1bd)
    dp = _deg_partials(ei3, ew2, zeros1)
    dinvp, t1p, s1p = _tc_scale(dp.reshape(_NC, _PK, 128), h1p)
    p1 = _edge_aggregate(t1p.reshape(_NP, _H), ei3, ew2, zeros16)
    t2p = _tc_mid(p1.reshape(_NC, _PK, 128), dinvp, s1p, b1t, Wmid)
    p2 = _edge_aggregate(t2p.reshape(_NP, _H), ei3, ew2, zeros16)
    op = _tc_combine(p2.reshape(_NC, _PK, 128), dinvp, t2p, b2t)
    return _tc_softmax(op.reshape(_NP, _H))


# speedup vs baseline: 1.1414x; 1.1414x over previous
import functools

import jax
import jax.numpy as jnp
from jax import lax
from jax.experimental import pallas as pl
from jax.experimental.pallas import tpu as pltpu
from jax.experimental.pallas import tpu_sc as plsc

_N = 10000
_E = 320000
_D = 128
_H = 16
_C = 8

_NC = 2
_NS = 16
_NW = _NC * _NS
_CHUNK = 128
_TCHUNK = _E // _CHUNK
_MAXCW = (_TCHUNK + _NW - 1) // _NW + 1
_NBUF = 5
_OUTER = (_MAXCW + _NBUF - 1) // _NBUF
_NP = 10240
_RPS1 = _NP // _NS
_PK = _NP // 8


def _sc_mesh():
    return plsc.VectorSubcoreMesh(core_axis_name="c", subcore_axis_name="s")


def _deg_partials(ei3, ew2, zeros1):
    @functools.partial(
        pl.kernel,
        out_type=jax.ShapeDtypeStruct((_NC, _NP, _H), jnp.float32),
        mesh=_sc_mesh(),
        scratch_types=[
            pltpu.VMEM((_MAXCW, _CHUNK), jnp.int32),
            pltpu.VMEM((_MAXCW, _CHUNK), jnp.float32),
            pltpu.VMEM((_RPS1,), jnp.float32),
            pltpu.VMEM((_RPS1, _H), jnp.float32),
            pltpu.VMEM_SHARED((_NP,), jnp.float32),
            pltpu.SemaphoreType.DMA,
        ],
        compiler_params=pltpu.CompilerParams(use_tc_tiling_on_sc=False),
    )
    def k(ei_hbm, ew_hbm, z_hbm, out_hbm, col_v, ew_v, degv, expb, acc, ssem):
        cid = lax.axis_index("c")
        sid = lax.axis_index("s")
        wid = cid * _NS + sid
        lo = wid * _TCHUNK // _NW
        ncw = (wid + 1) * _TCHUNK // _NW - lo
        r0 = sid * _RPS1
        pltpu.sync_copy(z_hbm.at[pl.ds(r0, _RPS1)], acc.at[pl.ds(r0, _RPS1)])
        pltpu.sync_copy(ei_hbm.at[pl.ds(lo, _MAXCW), 1], col_v)
        pltpu.sync_copy(ew_hbm.at[pl.ds(lo, _MAXCW)], ew_v)
        plsc.subcore_barrier()

        def body(c, carry):
            pltpu.async_copy(ew_v.at[c], acc.at[col_v.at[c]], ssem, add=True)
            return carry

        lax.fori_loop(0, ncw, body, 0)

        def drain(c, carry):
            pltpu.make_async_copy(ew_v.at[0], acc.at[col_v.at[0]], ssem).wait()
            return carry

        lax.fori_loop(0, ncw, drain, 0)
        plsc.subcore_barrier()
        pltpu.sync_copy(acc.at[pl.ds(r0, _RPS1)], degv)
        for g in range(_RPS1 // 16):
            dv = degv[pl.ds(g * 16, 16)]
            for j in range(16):
                expb[g * 16 + j] = jnp.broadcast_to(dv[j], (16,))
        pltpu.sync_copy(expb, out_hbm.at[cid, pl.ds(r0, _RPS1)])

    return k(ei3, ew2, zeros1)


def _edge_aggregate(table, ei3, ew2, zeros16):
    @functools.partial(
        pl.kernel,
        out_type=jax.ShapeDtypeStruct((_NC, _NP, _H), jnp.float32),
        mesh=_sc_mesh(),
        scratch_types=[
            pltpu.VMEM((_MAXCW, _CHUNK), jnp.int32),
            pltpu.VMEM((_MAXCW, _CHUNK), jnp.int32),
            pltpu.VMEM((_MAXCW, _CHUNK), jnp.float32),
            pltpu.VMEM((_NBUF, _CHUNK, _H), jnp.float32),
            pltpu.VMEM((_NBUF, _CHUNK, _H), jnp.float32),
            pltpu.VMEM_SHARED((_NP, _H), jnp.float32),
            pltpu.SemaphoreType.DMA((_NBUF,)),
            pltpu.SemaphoreType.DMA((_NBUF,)),
        ],
        compiler_params=pltpu.CompilerParams(use_tc_tiling_on_sc=False),
    )
    def k(h_hbm, ei_hbm, ew_hbm, z_hbm, out_hbm,
          row_v, col_v, ew_v, gbuf, sbuf, acc, gsem, ssem):
        cid = lax.axis_index("c")
        sid = lax.axis_index("s")
        wid = cid * _NS + sid
        lo = wid * _TCHUNK // _NW
        ncw = (wid + 1) * _TCHUNK // _NW - lo
        r0 = sid * _RPS1
        pltpu.sync_copy(z_hbm.at[pl.ds(r0, _RPS1)], acc.at[pl.ds(r0, _RPS1)])
        pltpu.sync_copy(ei_hbm.at[pl.ds(lo, _MAXCW), 0], row_v)
        pltpu.sync_copy(ei_hbm.at[pl.ds(lo, _MAXCW), 1], col_v)
        pltpu.sync_copy(ew_hbm.at[pl.ds(lo, _MAXCW)], ew_v)
        plsc.subcore_barrier()

        def gather(c, b):
            return pltpu.make_async_copy(
                h_hbm.at[row_v.at[c]], gbuf.at[b], gsem.at[b])

        def scatter(c, b):
            return pltpu.make_async_copy(
                sbuf.at[b], acc.at[col_v.at[c]], ssem.at[b])

        for b in range(_NBUF):
            gather(b, b).start()

        def outer(o, carry):
            for b in range(_NBUF):
                c = o * _NBUF + b

                @pl.when(c < ncw)
                def _():
                    gather(c, b).wait()

                @pl.when(jnp.logical_and(o > 0, c < ncw))
                def _():
                    scatter(c, b).wait()

                @pl.when(c < ncw)
                def _():
                    for g in range(_CHUNK // 16):
                        ewg = ew_v[c, pl.ds(g * 16, 16)]
                        for j in range(16):
                            e = g * 16 + j
                            sbuf[b, e] = gbuf[b, e] * ewg[j]
                    scatter(c, b).start(add=True)

                @pl.when(c + _NBUF < ncw)
                def _():
                    gather(c + _NBUF, b).start()
            return carry

        lax.fori_loop(0, _OUTER, outer, 0)
        for b in range(_NBUF):
            scatter(b, b).wait()
        plsc.subcore_barrier()
        pltpu.sync_copy(acc.at[pl.ds(r0, _RPS1)],
                        out_hbm.at[cid, pl.ds(r0, _RPS1)])

    return k(table, ei3, ew2, zeros16)


def _tc_matmul(xp, W1bd):
    def body(x_ref, w_ref, h_ref):
        hp = jnp.dot(x_ref[...], w_ref[...], preferred_element_type=jnp.float32)
        h_ref[...] = jnp.concatenate(
            [hp, jnp.zeros((_PK - _N // 8, 128), jnp.float32)], axis=0)

    return pl.pallas_call(
        body,
        out_shape=jax.ShapeDtypeStruct((_PK, 128), jnp.float32),
    )(xp, W1bd)


def _tc_scale(dp16, h1p):
    def body(dp_ref, h_ref, dinv_ref, t1_ref, s1_ref):
        deg = dp_ref[0] + dp_ref[1] + 1.0
        dinv = jnp.where(deg > 0, lax.rsqrt(jnp.where(deg > 0, deg, 1.0)), 0.0)
        h = h_ref[...]
        dinv_ref[...] = dinv
        t1_ref[...] = h * dinv
        s1_ref[...] = h * dinv * dinv

    return pl.pallas_call(
        body,
        out_shape=[
            jax.ShapeDtypeStruct((_PK, 128), jnp.float32),
            jax.ShapeDtypeStruct((_PK, 128), jnp.float32),
            jax.ShapeDtypeStruct((_PK, 128), jnp.float32),
        ],
    )(dp16, h1p)


def _tc_mid(p1p, dinvp, s1p, b1t, Wmid):
    def body(p_ref, d_ref, s_ref, b_ref, w_ref, out_ref):
        d = d_ref[...]
        v = jnp.maximum(d * (p_ref[0] + p_ref[1]) + s_ref[...] + b_ref[...],
                        0.0)
        out_ref[...] = d * jnp.dot(v, w_ref[...],
                                   preferred_element_type=jnp.float32)

    return pl.pallas_call(
        body,
        out_shape=jax.ShapeDtypeStruct((_PK, 128), jnp.float32),
    )(p1p, dinvp, s1p, b1t, Wmid)


def _tc_combine(p2p, dinvp, t2p, b2t):
    def body(p_ref, d_ref, t_ref, b_ref, out_ref):
        out_ref[...] = (d_ref[...] * (p_ref[0] + p_ref[1] + t_ref[...])
                        + b_ref[...])

    return pl.pallas_call(
        body,
        out_shape=jax.ShapeDtypeStruct((_PK, 128), jnp.float32),
    )(p2p, dinvp, t2p, b2t)


def _tc_softmax(o):
    def body(o_ref, out_ref):
        ot = o_ref[:_N, :_C].T
        m = jnp.max(ot, axis=0, keepdims=True)
        e = jnp.exp(ot - m)
        s = jnp.sum(e, axis=0, keepdims=True)
        out_ref[...] = ot - m - jnp.log(s)

    return pl.pallas_call(
        body,
        out_shape=jax.ShapeDtypeStruct((_C, _N), jnp.float32),
    )(o).T


def kernel(x, edge_index, edge_weight, W1, b1, W2, b2):
    f32 = jnp.float32
    ei3 = edge_index.reshape(2, _TCHUNK, _CHUNK).transpose(1, 0, 2)
    ew2 = edge_weight.astype(f32).reshape(_TCHUNK, _CHUNK)
    zeros1 = jnp.zeros((_NP,), f32)
    zeros16 = jnp.zeros((_NP, _H), f32)
    eye8 = jnp.eye(8, dtype=f32)
    W1bd = jnp.kron(eye8, W1)
    Wmid = jnp.kron(eye8, jnp.pad(W2, ((0, 0), (0, _H - _C))))
    b1t = jnp.tile(b1, 8).reshape(1, 128)
    b2t = jnp.tile(jnp.pad(b2, (0, _H - _C)), 8).reshape(1, 128)

    h1p = _tc_matmul(x.reshape(_N // 8, 8 * _D), W1bd)
    dp = _deg_partials(ei3, ew2, zeros1)
    dinvp, t1p, s1p = _tc_scale(dp.reshape(_NC, _PK, 128), h1p)
    p1 = _edge_aggregate(t1p.reshape(_NP, _H), ei3, ew2, zeros16)
    t2p = _tc_mid(p1.reshape(_NC, _PK, 128), dinvp, s1p, b1t, Wmid)
    p2 = _edge_aggregate(t2p.reshape(_NP, _H), ei3, ew2, zeros16)
    op = _tc_combine(p2.reshape(_NC, _PK, 128), dinvp, t2p, b2t)
    return _tc_softmax(op.reshape(_NP, _H))
